# Initial kernel scaffold; baseline (speedup 1.0000x reference)
import jax
import jax.numpy as jnp
from jax.experimental import pallas as pl


def kernel(x, edge_index, W1, b1, W2, b2):
    raise NotImplementedError("write your pallas kernel here")


# speedup vs baseline: 28.3754x; 28.3754x over previous
import functools

import jax
import jax.numpy as jnp
from jax import lax
from jax.experimental import pallas as pl
from jax.experimental.pallas import tpu as pltpu
from jax.experimental.pallas import tpu_sc as plsc

N = 10000
E = 320000
DIN = 128
DH = 16
DO = 2

NC = 2
NS = 16
NW = NC * NS
ET = E // NW
S = 80
NB = ET // S
NP = 10240
RT = NP // NS

@functools.cache
def _sc_kernels():
    mesh = plsc.VectorSubcoreMesh(
        core_axis_name="c", subcore_axis_name="s", num_cores=NC, num_subcores=NS
    )

    @functools.partial(
        pl.kernel,
        out_type=jax.ShapeDtypeStruct((NC, NP, 8), jnp.float32),
        mesh=mesh,
        compiler_params=pltpu.CompilerParams(use_tc_tiling_on_sc=False),
        scratch_types=[
            pltpu.VMEM((NB, S), jnp.int32),
            pltpu.VMEM((S, 8), jnp.float32),
            pltpu.VMEM_SHARED((NP, 8), jnp.float32),
        ],
    )
    def _sc_deg(dst_hbm, ones_hbm, zeros_hbm, part_hbm, didx_v, ones_v, acc_sh):
        cid = lax.axis_index("c")
        sid = lax.axis_index("s")
        tid = cid * NS + sid

        pltpu.sync_copy(zeros_hbm, acc_sh.at[pl.ds(sid * RT, RT)])
        pltpu.sync_copy(dst_hbm.at[tid], didx_v)
        pltpu.sync_copy(ones_hbm, ones_v)
        plsc.subcore_barrier()

        def body(j, carry):
            pltpu.sync_copy(ones_v, acc_sh.at[didx_v.at[j]], add=True)
            return carry

        lax.fori_loop(0, NB, body, 0)
        plsc.subcore_barrier()
        pltpu.sync_copy(
            acc_sh.at[pl.ds(sid * RT, RT)],
            part_hbm.at[cid, pl.ds(sid * RT, RT)],
        )

    @functools.partial(
        pl.kernel,
        out_type=jax.ShapeDtypeStruct((NC, NP, DH), jnp.float32),
        mesh=mesh,
        compiler_params=pltpu.CompilerParams(use_tc_tiling_on_sc=False),
        scratch_types=[
            pltpu.VMEM((NB, S), jnp.int32),
            pltpu.VMEM((NB, S), jnp.int32),
            pltpu.VMEM((S, DH), jnp.float32),
            pltpu.VMEM_SHARED((NP, DH), jnp.float32),
            pltpu.SemaphoreType.DMA,
        ],
    )
    def _sc_edge_pass(g_hbm, src_hbm, dst_hbm, zeros_hbm, part_hbm,
                      sidx_v, didx_v, rows_v, acc_sh, sem):
        cid = lax.axis_index("c")
        sid = lax.axis_index("s")
        tid = cid * NS + sid

        pltpu.sync_copy(zeros_hbm, acc_sh.at[pl.ds(sid * RT, RT)])
        pltpu.sync_copy(src_hbm.at[tid], sidx_v)
        pltpu.sync_copy(dst_hbm.at[tid], didx_v)
        plsc.subcore_barrier()

        def body(j, carry):
            pltpu.async_copy(g_hbm.at[sidx_v.at[j]], rows_v, sem).wait()
            pltpu.sync_copy(rows_v, acc_sh.at[didx_v.at[j]], add=True)
            return carry

        lax.fori_loop(0, NB, body, 0)
        plsc.subcore_barrier()
        pltpu.sync_copy(
            acc_sh.at[pl.ds(sid * RT, RT)],
            part_hbm.at[cid, pl.ds(sid * RT, RT)],
        )

    return _sc_deg, _sc_edge_pass


_R = 1000


def _tc_prep_body(x_ref, w1_ref, degp_ref, g1_ref, dinvb_ref):
    h = jnp.dot(x_ref[...], w1_ref[...], preferred_element_type=jnp.float32)
    deg = degp_ref[0, :, 0:1] + degp_ref[1, :, 0:1] + 1.0
    dinv = lax.rsqrt(deg)
    g1_ref[...] = h * dinv
    dinvb_ref[...] = jnp.broadcast_to(dinv, (_R, DH))


def _tc_prep(x, w1, degp):
    return pl.pallas_call(
        _tc_prep_body,
        grid=(N // _R,),
        in_specs=[
            pl.BlockSpec((_R, DIN), lambda i: (i, 0)),
            pl.BlockSpec((DIN, DH), lambda i: (0, 0)),
            pl.BlockSpec((NC, _R, 8), lambda i: (0, i, 0)),
        ],
        out_specs=[
            pl.BlockSpec((_R, DH), lambda i: (i, 0)),
            pl.BlockSpec((_R, DH), lambda i: (i, 0)),
        ],
        out_shape=[
            jax.ShapeDtypeStruct((N, DH), jnp.float32),
            jax.ShapeDtypeStruct((N, DH), jnp.float32),
        ],
    )(x, w1, degp)


def _tc_mid_body(p_ref, g1_ref, dinvb_ref, b1_ref, g2_ref):
    agg = (p_ref[0] + p_ref[1] + g1_ref[...]) * dinvb_ref[...]
    r = jnp.maximum(agg + b1_ref[...], 0.0)
    g2_ref[...] = r * dinvb_ref[...]


def _tc_mid(p, g1, dinvb, b1):
    return pl.pallas_call(
        _tc_mid_body,
        grid=(N // _R,),
        in_specs=[
            pl.BlockSpec((NC, _R, DH), lambda i: (0, i, 0)),
            pl.BlockSpec((_R, DH), lambda i: (i, 0)),
            pl.BlockSpec((_R, DH), lambda i: (i, 0)),
            pl.BlockSpec((1, DH), lambda i: (0, 0)),
        ],
        out_specs=pl.BlockSpec((_R, DH), lambda i: (i, 0)),
        out_shape=jax.ShapeDtypeStruct((N, DH), jnp.float32),
    )(p, g1, dinvb, b1)


def _tc_final_body(q_ref, g2_ref, dinvb_ref, w2_ref, b2_ref, out_ref):
    agg = (q_ref[0] + q_ref[1] + g2_ref[...]) * dinvb_ref[...]
    logits = jnp.dot(agg, w2_ref[...], preferred_element_type=jnp.float32)
    logits = logits + b2_ref[...]
    m = jnp.max(logits, axis=1, keepdims=True)
    sh = logits - m
    out_ref[...] = sh - jnp.log(jnp.sum(jnp.exp(sh), axis=1, keepdims=True))


def _tc_final(q, g2, dinvb, w2, b2):
    return pl.pallas_call(
        _tc_final_body,
        grid=(N // _R,),
        in_specs=[
            pl.BlockSpec((NC, _R, DH), lambda i: (0, i, 0)),
            pl.BlockSpec((_R, DH), lambda i: (i, 0)),
            pl.BlockSpec((_R, DH), lambda i: (i, 0)),
            pl.BlockSpec((DH, DO), lambda i: (0, 0)),
            pl.BlockSpec((1, DO), lambda i: (0, 0)),
        ],
        out_specs=pl.BlockSpec((_R, DO), lambda i: (i, 0)),
        out_shape=jax.ShapeDtypeStruct((N, DO), jnp.float32),
    )(q, g2, dinvb, w2, b2)


def kernel(x, edge_index, W1, b1, W2, b2):
    src = edge_index[0].astype(jnp.int32).reshape(NW, NB, S)
    dst = edge_index[1].astype(jnp.int32).reshape(NW, NB, S)

    ones8 = jnp.ones((S, 8), jnp.float32)
    zeros8 = jnp.zeros((RT, 8), jnp.float32)
    zeros16 = jnp.zeros((RT, DH), jnp.float32)

    sc_deg, sc_edge_pass = _sc_kernels()
    degp = sc_deg(dst, ones8, zeros8)
    g1, dinvb = _tc_prep(x, W1, degp)
    p1 = sc_edge_pass(g1, src, dst, zeros16)
    g2 = _tc_mid(p1, g1, dinvb, b1.reshape(1, DH))
    p2 = sc_edge_pass(g2, src, dst, zeros16)
    return _tc_final(p2, g2, dinvb, W2, b2.reshape(1, DO))
